# baseline (speedup 1.0000x reference)
import functools

import jax
import jax.numpy as jnp
from jax import lax
from jax.experimental import pallas as pl
from jax.experimental.pallas import tpu as pltpu
from jax.experimental.pallas import tpu_sc as plsc


@functools.lru_cache(maxsize=None)
def _make_embed(B, D, V, CV):
    info = plsc.get_sparse_core_info()
    NC, NS = info.num_cores, info.num_subcores
    L = info.num_lanes
    NW = NC * NS
    b_per_w = B // NW
    C = 32
    CG = C - CV
    n_chunks = b_per_w // C
    mesh = plsc.VectorSubcoreMesh(core_axis_name="c", subcore_axis_name="s")

    @functools.partial(
        pl.kernel,
        mesh=mesh,
        compiler_params=pltpu.CompilerParams(needs_layout_passes=False),
        out_type=jax.ShapeDtypeStruct((B, D), jnp.float32),
        scratch_types=[
            pltpu.VMEM((V * D,), jnp.float32),
            pltpu.VMEM((b_per_w,), jnp.int32),
            pltpu.VMEM((b_per_w,), jnp.int32),
            pltpu.VMEM((2, C, D), jnp.float32),
            pltpu.SemaphoreType.DMA,
            pltpu.SemaphoreType.DMA,
            pltpu.SemaphoreType.DMA,
            pltpu.SemaphoreType.DMA,
        ],
    )
    def k(rep_hbm, flat_hbm, idx_hbm, out_hbm, tbl_v, idx_v, idx_r, rows_v, g0, g1, w0, w1):
        wid = lax.axis_index("s") * NC + lax.axis_index("c")
        base = wid * b_per_w
        gsem = [g0, g1]
        wsem = [w0, w1]
        pltpu.sync_copy(flat_hbm, tbl_v)
        pltpu.sync_copy(idx_hbm.at[pl.ds(base, b_per_w)], idx_v)
        iota = lax.iota(jnp.int32, L)
        off = (wid * V).astype(jnp.int32)
        for j in range(b_per_w // L):
            sl = pl.ds(j * L, L)
            idx_r[sl] = idx_v[sl] + off

        def assemble(i, b, cv):
            @plsc.parallel_loop(0, cv, step=1, unroll=4)
            def row_body(j):
                jm = j % L
                seg16 = idx_v[pl.ds(i * C + j - jm, L)]
                seg_s = jnp.max(jnp.where(iota == jm, seg16, 0))
                rbase = seg_s * D

                @plsc.parallel_loop(0, D, step=L, unroll=16)
                def copy_body(g):
                    rows_v[b, j, pl.ds(g, L)] = tbl_v[pl.ds(rbase + g, L)]

        @pl.loop(0, n_chunks, step=2)
        def chunk_loop(i0):
            for b in range(2):
                i = i0 + b
                cv = CV + 8 * b
                cg = C - cv

                @pl.when(i >= 2)
                def _():
                    pltpu.make_async_copy(
                        rows_v.at[b], out_hbm.at[pl.ds(0, C)], wsem[b]
                    ).wait()

                if cg:
                    pltpu.async_copy(
                        rep_hbm.at[idx_r.at[pl.ds(i * C + cv, cg)]],
                        rows_v.at[b].at[pl.ds(cv, cg)],
                        gsem[b],
                    )
                assemble(i, b, cv)
                if cg:
                    pltpu.make_async_copy(
                        rep_hbm.at[pl.ds(0, cg)],
                        rows_v.at[b].at[pl.ds(cv, cg)],
                        gsem[b],
                    ).wait()
                pltpu.async_copy(
                    rows_v.at[b],
                    out_hbm.at[pl.ds(base + i * C, C)],
                    wsem[b],
                )

        for b in range(2):
            pltpu.make_async_copy(
                rows_v.at[b], out_hbm.at[pl.ds(0, C)], wsem[b]
            ).wait()

    return k


def kernel(segment_ids, table):
    B = segment_ids.shape[0] * segment_ids.shape[1]
    V, D = table.shape
    NW = 32
    CV = 24
    idx_flat = segment_ids.reshape(B).astype(jnp.int32)
    rep_table = jnp.broadcast_to(table, (NW, V, D)).reshape(NW * V, D)
    out = _make_embed(B, D, V, CV)(rep_table, table.reshape(V * D), idx_flat)
    return out.reshape(segment_ids.shape + (D,))


# speedup vs baseline: 1.1659x; 1.1659x over previous
import functools

import jax
import jax.numpy as jnp
from jax import lax
from jax.experimental import pallas as pl
from jax.experimental.pallas import tpu as pltpu
from jax.experimental.pallas import tpu_sc as plsc


@functools.lru_cache(maxsize=None)
def _make_embed(B, D, V, CV):
    info = plsc.get_sparse_core_info()
    NC, NS = info.num_cores, info.num_subcores
    L = info.num_lanes
    NW = NC * NS
    b_per_w = B // NW
    C = 32
    CG = C - CV
    n_chunks = b_per_w // C
    mesh = plsc.VectorSubcoreMesh(core_axis_name="c", subcore_axis_name="s")

    @functools.partial(
        pl.kernel,
        mesh=mesh,
        compiler_params=pltpu.CompilerParams(needs_layout_passes=False),
        out_type=jax.ShapeDtypeStruct((B, D), jnp.float32),
        scratch_types=[
            pltpu.VMEM((V * D,), jnp.float32),
            pltpu.VMEM((b_per_w,), jnp.int32),
            pltpu.VMEM((b_per_w,), jnp.int32),
            pltpu.VMEM((2, C, D), jnp.float32),
            pltpu.VMEM((C * 16, ), jnp.int32),
            pltpu.SemaphoreType.DMA,
            pltpu.SemaphoreType.DMA,
            pltpu.SemaphoreType.DMA,
            pltpu.SemaphoreType.DMA,
        ],
    )
    def k(rep_hbm, flat_hbm, idx_hbm, out_hbm, tbl_v, idx_v, idx_r, rows_v, spl_v, g0, g1, w0, w1):
        wid = lax.axis_index("s") * NC + lax.axis_index("c")
        base = wid * b_per_w
        gsem = [g0, g1]
        wsem = [w0, w1]
        pltpu.sync_copy(flat_hbm, tbl_v)
        pltpu.sync_copy(idx_hbm.at[pl.ds(base, b_per_w)], idx_v)
        iota = lax.iota(jnp.int32, L)
        off = (wid * V).astype(jnp.int32)
        for j in range(b_per_w // L):
            sl = pl.ds(j * L, L)
            idx_r[sl] = idx_v[sl] + off

        SEC = 128
        NG = SEC // L

        def assemble(i, b, cv):
            @plsc.parallel_loop(0, cv, step=1, unroll=4)
            def splat_body(j):
                spl_v[pl.ds(j * L, L)] = plsc.load_gather(
                    idx_v, [jnp.full((L,), i * C + j, jnp.int32)]
                )

            for sec in range(D // SEC):
                tsec = [
                    [
                        tbl_v[pl.ds(r * D + sec * SEC + g * L, L)]
                        for g in range(NG)
                    ]
                    for r in range(V)
                ]

                @plsc.parallel_loop(0, cv, step=1, unroll=2)
                def row_body(j):
                    spl = spl_v[pl.ds(j * L, L)]
                    for g in range(NG):
                        v = tsec[V - 1][g]
                        for r in range(V - 2, -1, -1):
                            v = jnp.where(spl == r, tsec[r][g], v)
                        rows_v[b, j, pl.ds(sec * SEC + g * L, L)] = v

        @pl.loop(0, n_chunks, step=2)
        def chunk_loop(i0):
            for b in range(2):
                i = i0 + b
                cv = CV
                cg = C - cv

                @pl.when(i >= 2)
                def _():
                    pltpu.make_async_copy(
                        rows_v.at[b], out_hbm.at[pl.ds(0, C)], wsem[b]
                    ).wait()

                if cg:
                    pltpu.async_copy(
                        rep_hbm.at[idx_r.at[pl.ds(i * C + cv, cg)]],
                        rows_v.at[b].at[pl.ds(cv, cg)],
                        gsem[b],
                    )
                assemble(i, b, cv)
                if cg:
                    pltpu.make_async_copy(
                        rep_hbm.at[pl.ds(0, cg)],
                        rows_v.at[b].at[pl.ds(cv, cg)],
                        gsem[b],
                    ).wait()
                pltpu.async_copy(
                    rows_v.at[b],
                    out_hbm.at[pl.ds(base + i * C, C)],
                    wsem[b],
                )

        for b in range(2):
            pltpu.make_async_copy(
                rows_v.at[b], out_hbm.at[pl.ds(0, C)], wsem[b]
            ).wait()

    return k


def kernel(segment_ids, table):
    B = segment_ids.shape[0] * segment_ids.shape[1]
    V, D = table.shape
    NW = 32
    CV = 32
    idx_flat = segment_ids.reshape(B).astype(jnp.int32)
    rep_table = jnp.broadcast_to(table, (NW, V, D)).reshape(NW * V, D)
    out = _make_embed(B, D, V, CV)(rep_table, table.reshape(V * D), idx_flat)
    return out.reshape(segment_ids.shape + (D,))


# speedup vs baseline: 1.1972x; 1.0269x over previous
import functools

import jax
import jax.numpy as jnp
from jax import lax
from jax.experimental import pallas as pl
from jax.experimental.pallas import tpu as pltpu
from jax.experimental.pallas import tpu_sc as plsc


@functools.lru_cache(maxsize=None)
def _make_embed(B, D, V):
    info = plsc.get_sparse_core_info()
    NC, NS = info.num_cores, info.num_subcores
    L = info.num_lanes
    NW = NC * NS
    b_per_w = B // NW
    C = 32
    n_chunks = b_per_w // C
    SEC = 128
    NG = SEC // L
    mesh = plsc.VectorSubcoreMesh(core_axis_name="c", subcore_axis_name="s")

    @functools.partial(
        pl.kernel,
        mesh=mesh,
        compiler_params=pltpu.CompilerParams(needs_layout_passes=False),
        out_type=jax.ShapeDtypeStruct((B, D), jnp.float32),
        scratch_types=[
            pltpu.VMEM((V * D,), jnp.float32),
            pltpu.VMEM((b_per_w,), jnp.int32),
            pltpu.VMEM((2, C, D), jnp.float32),
            pltpu.VMEM((C * 16,), jnp.int32),
            pltpu.SemaphoreType.DMA,
            pltpu.SemaphoreType.DMA,
        ],
    )
    def k(flat_hbm, idx_hbm, out_hbm, tbl_v, idx_v, rows_v, spl_v, w0, w1):
        wid = lax.axis_index("s") * NC + lax.axis_index("c")
        base = wid * b_per_w
        wsem = [w0, w1]
        pltpu.sync_copy(flat_hbm, tbl_v)
        pltpu.sync_copy(idx_hbm.at[pl.ds(base, b_per_w)], idx_v)

        def assemble(i, b):
            @plsc.parallel_loop(0, C, step=1, unroll=4)
            def splat_body(j):
                spl_v[pl.ds(j * L, L)] = plsc.load_gather(
                    idx_v, [jnp.full((L,), i * C + j, jnp.int32)]
                )

            for sec in range(D // SEC):
                tsec = [
                    [
                        tbl_v[pl.ds(r * D + sec * SEC + g * L, L)]
                        for g in range(NG)
                    ]
                    for r in range(V)
                ]

                @plsc.parallel_loop(0, C, step=1, unroll=2)
                def row_body(j):
                    spl = spl_v[pl.ds(j * L, L)]
                    for g in range(NG):
                        v = tsec[V - 1][g]
                        for r in range(V - 2, -1, -1):
                            v = jnp.where(spl == r, tsec[r][g], v)
                        rows_v[b, j, pl.ds(sec * SEC + g * L, L)] = v

        @pl.loop(0, n_chunks, step=2)
        def chunk_loop(i0):
            for b in range(2):
                i = i0 + b

                @pl.when(i >= 2)
                def _():
                    pltpu.make_async_copy(
                        rows_v.at[b], out_hbm.at[pl.ds(0, C)], wsem[b]
                    ).wait()

                assemble(i, b)
                pltpu.async_copy(
                    rows_v.at[b],
                    out_hbm.at[pl.ds(base + i * C, C)],
                    wsem[b],
                )

        for b in range(2):
            pltpu.make_async_copy(
                rows_v.at[b], out_hbm.at[pl.ds(0, C)], wsem[b]
            ).wait()

    return k


def kernel(segment_ids, table):
    B = segment_ids.shape[0] * segment_ids.shape[1]
    V, D = table.shape
    idx_flat = segment_ids.reshape(B).astype(jnp.int32)
    out = _make_embed(B, D, V)(table.reshape(V * D), idx_flat)
    return out.reshape(segment_ids.shape + (D,))
